# baseline (speedup 1.0000x reference)
import functools

import jax
import jax.numpy as jnp
from jax import lax
from jax.experimental import pallas as pl
from jax.experimental.pallas import tpu as pltpu
from jax.experimental.pallas import tpu_sc as plsc

L = 200
B = 4096
H = 64
NC, NS = 2, 16
NW = NC * NS
BPW = B // NW
S0, S1 = 96, 104
UNROLL = 8


def _sc_embed_sum(features_t_flat, table):
    mesh = plsc.VectorSubcoreMesh(core_axis_name="c", subcore_axis_name="s")

    @functools.partial(
        pl.kernel,
        mesh=mesh,
        out_type=jax.ShapeDtypeStruct((B, H), jnp.float32),
        compiler_params=pltpu.CompilerParams(use_tc_tiling_on_sc=False),
        scratch_types=[
            pltpu.VMEM((BPW * L,), jnp.int32),
            pltpu.VMEM((L, H), jnp.float32),
            pltpu.VMEM((L, H), jnp.float32),
            pltpu.VMEM((BPW, H), jnp.float32),
            pltpu.SemaphoreType.DMA,
            pltpu.SemaphoreType.DMA,
        ],
    )
    def k(feat_hbm, table_hbm, out_hbm, idx_v, rows0, rows1, out_v, sem0, sem1):
        wid = lax.axis_index("s") * NC + lax.axis_index("c")
        base = wid * BPW

        pltpu.sync_copy(feat_hbm.at[pl.ds(base * L, BPW * L)], idx_v)

        def copies(b, rows, sem):
            off = b * L
            c1 = pltpu.make_async_copy(
                table_hbm.at[idx_v.at[pl.ds(off, S0)]],
                rows.at[pl.ds(0, S0)], sem)
            c2 = pltpu.make_async_copy(
                table_hbm.at[idx_v.at[pl.ds(off + S0, S1)]],
                rows.at[pl.ds(S0, S1)], sem)
            return c1, c2

        def fire(b, rows, sem):
            c1, c2 = copies(b, rows, sem)
            c1.start()
            c2.start()

        def drain(b, rows, sem):
            c1, c2 = copies(b, rows, sem)
            c1.wait()
            c2.wait()

        def accumulate(rows):
            zero = jnp.zeros((16,), jnp.float32)

            def body(j, accs):
                a0, a1, a2, a3 = accs
                for u in range(UNROLL):
                    l = j * UNROLL + u
                    a0 = a0 + rows[l, pl.ds(0, 16)]
                    a1 = a1 + rows[l, pl.ds(16, 16)]
                    a2 = a2 + rows[l, pl.ds(32, 16)]
                    a3 = a3 + rows[l, pl.ds(48, 16)]
                return a0, a1, a2, a3

            return lax.fori_loop(0, L // UNROLL, body, (zero, zero, zero, zero))

        fire(0, rows0, sem0)
        fire(1, rows1, sem1)

        def outer(i, _):
            for phase, (rows, sem) in enumerate(((rows0, sem0), (rows1, sem1))):
                b = 2 * i + phase
                drain(b, rows, sem)
                a0, a1, a2, a3 = accumulate(rows)
                nb = b + 2

                @pl.when(nb < BPW)
                def _():
                    fire(nb, rows, sem)

                out_v[b, pl.ds(0, 16)] = a0
                out_v[b, pl.ds(16, 16)] = a1
                out_v[b, pl.ds(32, 16)] = a2
                out_v[b, pl.ds(48, 16)] = a3
            return 0

        lax.fori_loop(0, BPW // 2, outer, 0)
        pltpu.sync_copy(out_v, out_hbm.at[pl.ds(base, BPW)])

    return k(features_t_flat, table)


def _tc_head(h, W, b2):
    def body(h_ref, w_ref, b_ref, o_ref):
        hv = jnp.maximum(h_ref[...], 0.0)
        o_ref[...] = lax.dot_general(
            hv, w_ref[...], (((1,), (1,)), ((), ())),
            preferred_element_type=jnp.float32) + b_ref[...]

    return pl.pallas_call(
        body,
        out_shape=jax.ShapeDtypeStruct((B, H), jnp.float32),
    )(h, W, b2)


def kernel(features, table, W, b):
    feat_t = jnp.transpose(features).reshape(-1)
    pooled = _sc_embed_sum(feat_t, table)
    return _tc_head(pooled, W, b.reshape(1, H))
